# baseline (speedup 1.0000x reference)
import functools

import jax
import jax.numpy as jnp
import numpy as np
from jax.experimental import pallas as pl
from jax.experimental.pallas import tpu as pltpu


def _yunet_body(x_ref, w1_ref, b1_ref, w2_ref, b2_ref, w3_ref, b3_ref,
                wh_ref, bh_ref, wb_ref, bb_ref, wk_ref, bk_ref,
                cls_ref, bbox_ref, obj_ref, kps_ref,
                col_ref, *, H, W, BC):
    HW = H * W
    B = BC // 3

    lane = jax.lax.broadcasted_iota(jnp.int32, (1, HW), 1)
    row = lane // W
    col = lane % W

    taps = []
    for di in range(3):
        for dj in range(3):
            s = (di - 1) * W + (dj - 1)
            amount = (-s) % HW
            valid = ((row + (di - 1) >= 0) & (row + (di - 1) < H) &
                     (col + (dj - 1) >= 0) & (col + (dj - 1) < W))
            taps.append((amount, valid))

    def conv3x3_relu(x, w_ref, b_ref):
        for t, (amount, valid) in enumerate(taps):
            shifted = x if amount == 0 else pltpu.roll(x, amount, axis=1)
            col_ref[t * BC:(t + 1) * BC, :] = jnp.where(
                valid, shifted, 0.0).astype(jnp.bfloat16)
        y = jnp.dot(w_ref[...], col_ref[...],
                    preferred_element_type=jnp.float32) + b_ref[...]
        return jnp.maximum(y, 0.0)

    x = x_ref[...]
    x = conv3x3_relu(x, w1_ref, b1_ref)
    x = conv3x3_relu(x, w2_ref, b2_ref)
    f = conv3x3_relu(x, w3_ref, b3_ref)

    fb = f.astype(jnp.bfloat16)
    y = jnp.dot(wh_ref[...], fb,
                preferred_element_type=jnp.float32) + bh_ref[...]
    cls_ref[...] = y[0:B].reshape(B * H, W)
    obj_ref[...] = y[B:2 * B].reshape(B * H, W)

    f3 = fb.reshape(16 * B * H, W)
    a = jnp.concatenate(
        [f3[ci * B * H:(ci + 1) * B * H] for ci in range(16)],
        axis=1)
    yb = jnp.dot(a, wb_ref[...],
                 preferred_element_type=jnp.float32) + bb_ref[...]
    bbox_ref[...] = yb.reshape(B * HW * 4 // 128, 128)
    yk = jnp.dot(a, wk_ref[...],
                 preferred_element_type=jnp.float32) + bk_ref[...]
    kps_ref[...] = yk.reshape(B * HW * 10 // 128, 128)


def kernel(img, dn_w, dn_b, lle_w, lle_b, bb_w, bb_b, hd_w, hd_b):
    n, c, h, w = img.shape
    hw = h * w
    B = 8 if n % 8 == 0 else 1
    eye = jnp.eye(B, dtype=jnp.float32)

    x = img.astype(jnp.float32).reshape(n * c, hw)

    def conv_w_big(wt):
        o, i = wt.shape[0], wt.shape[1]
        wr = jnp.transpose(wt, (2, 3, 0, 1)).reshape(9, o, i)
        big = jnp.einsum('ab,toc->aotbc', eye, wr)
        return big.reshape(B * o, 9 * B * i)

    def conv_b_big(bt):
        return jnp.tile(bt, B).reshape(-1, 1)

    perm3 = np.asarray([(j % B) * 16 + j // B for j in range(16 * B)])

    wh_big = jnp.einsum('ab,oc->aobc', eye, hd_w).reshape(B * 16, B * 16)
    bh_big = jnp.tile(hd_b, B).reshape(-1, 1)
    perm = np.asarray([b * 16 + 0 for b in range(B)] +
                      [b * 16 + 5 for b in range(B)])
    wh_big = wh_big[perm][:, perm3]
    bh_big = bh_big[perm]

    jb = np.arange(4 * w)
    kb = np.zeros((4, w, 4 * w), np.float32)
    kb[jb % 4, jb // 4, jb] = 1.0
    wb_exp = jnp.dot(hd_w[1:5].T, jnp.asarray(kb.reshape(4, -1)))
    wb_exp = wb_exp.reshape(16 * w, 4 * w)
    bb_row = jnp.tile(hd_b[1:5], w).reshape(1, 4 * w)
    jk = np.arange(10 * w)
    kk = np.zeros((10, w, 10 * w), np.float32)
    kk[jk % 10, jk // 10, jk] = 1.0
    wk_exp = jnp.dot(hd_w[6:16].T, jnp.asarray(kk.reshape(10, -1)))
    wk_exp = wk_exp.reshape(16 * w, 10 * w)
    bk_row = jnp.tile(hd_b[6:16], w).reshape(1, 10 * w)

    operands = (
        x,
        conv_w_big(dn_w).astype(jnp.bfloat16), conv_b_big(dn_b),
        conv_w_big(lle_w).astype(jnp.bfloat16), conv_b_big(lle_b),
        conv_w_big(bb_w)[perm3].astype(jnp.bfloat16), conv_b_big(bb_b)[perm3],
        wh_big.astype(jnp.bfloat16), bh_big,
        wb_exp.astype(jnp.bfloat16), bb_row,
        wk_exp.astype(jnp.bfloat16), bk_row,
    )
    in_specs = [pl.BlockSpec((B * c, hw), lambda i: (i, 0))]
    in_specs += [pl.BlockSpec(op.shape, lambda i: (0, 0))
                 for op in operands[1:]]

    out_shapes = (
        jax.ShapeDtypeStruct((n * h, w), jnp.float32),
        jax.ShapeDtypeStruct((n * hw * 4 // 128, 128), jnp.float32),
        jax.ShapeDtypeStruct((n * h, w), jnp.float32),
        jax.ShapeDtypeStruct((n * hw * 10 // 128, 128), jnp.float32),
    )
    out_specs = (
        pl.BlockSpec((B * h, w), lambda i: (i, 0)),
        pl.BlockSpec((B * hw * 4 // 128, 128), lambda i: (i, 0)),
        pl.BlockSpec((B * h, w), lambda i: (i, 0)),
        pl.BlockSpec((B * hw * 10 // 128, 128), lambda i: (i, 0)),
    )

    cls2, bbox2, obj2, kps2 = pl.pallas_call(
        functools.partial(_yunet_body, H=h, W=w, BC=B * c),
        out_shape=out_shapes,
        grid=(n // B,),
        in_specs=in_specs,
        out_specs=out_specs,
        scratch_shapes=[pltpu.VMEM((9 * B * c, hw), jnp.bfloat16)],
        compiler_params=pltpu.CompilerParams(
            dimension_semantics=("parallel",)),
    )(*operands)

    cls_p = cls2.reshape(n, hw, 1)
    bbox_p = bbox2.reshape(n, hw, 4)
    obj_p = obj2.reshape(n, hw, 1)
    kps_p = kps2.reshape(n, hw, 10)
    return cls_p, bbox_p, obj_p, kps_p


# speedup vs baseline: 9.5048x; 9.5048x over previous
import functools

import jax
import jax.numpy as jnp
from jax.experimental import pallas as pl
from jax.experimental.pallas import tpu as pltpu


def _yunet_body(x_ref, w1_ref, b1_ref, w2_ref, b2_ref, w3_ref, b3_ref,
                wh_ref, bh_ref,
                cls_ref, bbox_ref, obj_ref, kps_ref,
                col_ref, *, H, W, BC):
    HW = H * W
    B = BC // 3

    lane = jax.lax.broadcasted_iota(jnp.int32, (1, HW), 1)
    row = lane // W
    col = lane % W

    taps = []
    for di in range(3):
        for dj in range(3):
            s = (di - 1) * W + (dj - 1)
            amount = (-s) % HW
            valid = ((row + (di - 1) >= 0) & (row + (di - 1) < H) &
                     (col + (dj - 1) >= 0) & (col + (dj - 1) < W))
            taps.append((amount, valid))

    def conv3x3_relu(x, w_ref, b_ref):
        for t, (amount, valid) in enumerate(taps):
            shifted = x if amount == 0 else pltpu.roll(x, amount, axis=1)
            col_ref[t * BC:(t + 1) * BC, :] = jnp.where(
                valid, shifted, 0.0).astype(jnp.bfloat16)
        y = jnp.dot(w_ref[...], col_ref[...],
                    preferred_element_type=jnp.float32) + b_ref[...]
        return jnp.maximum(y, 0.0)

    x = x_ref[...]
    x = conv3x3_relu(x, w1_ref, b1_ref)
    x = conv3x3_relu(x, w2_ref, b2_ref)
    f = conv3x3_relu(x, w3_ref, b3_ref)

    y = jnp.dot(wh_ref[...], f.astype(jnp.bfloat16),
                preferred_element_type=jnp.float32) + bh_ref[...]
    cls_ref[...] = y[0:B].reshape(B * H, W)
    bbox_ref[...] = y[B:5 * B]
    obj_ref[...] = y[5 * B:6 * B].reshape(B * H, W)
    kps_ref[...] = y[6 * B:16 * B]


def kernel(img, dn_w, dn_b, lle_w, lle_b, bb_w, bb_b, hd_w, hd_b):
    n, c, h, w = img.shape
    hw = h * w
    B = 8 if n % 8 == 0 else 1
    eye = jnp.eye(B, dtype=jnp.float32)

    x = img.astype(jnp.float32).reshape(n * c, hw)

    def conv_w_big(wt):
        o, i = wt.shape[0], wt.shape[1]
        wr = jnp.transpose(wt, (2, 3, 0, 1)).reshape(9, o, i)
        big = jnp.einsum('ab,toc->aotbc', eye, wr)
        return big.reshape(B * o, 9 * B * i)

    def conv_b_big(bt):
        return jnp.tile(bt, B).reshape(-1, 1)

    wh_big = jnp.einsum('ab,oc->aobc', eye, hd_w).reshape(B * 16, B * 16)
    bh_big = jnp.tile(hd_b, B).reshape(-1, 1)
    perm = ([b * 16 + 0 for b in range(B)] +
            [b * 16 + 1 + i for b in range(B) for i in range(4)] +
            [b * 16 + 5 for b in range(B)] +
            [b * 16 + 6 + i for b in range(B) for i in range(10)])
    perm = jnp.asarray(perm, dtype=jnp.int32)
    wh_big = wh_big[perm]
    bh_big = bh_big[perm]

    operands = (
        x,
        conv_w_big(dn_w).astype(jnp.bfloat16), conv_b_big(dn_b),
        conv_w_big(lle_w).astype(jnp.bfloat16), conv_b_big(lle_b),
        conv_w_big(bb_w).astype(jnp.bfloat16), conv_b_big(bb_b),
        wh_big.astype(jnp.bfloat16), bh_big,
    )
    in_specs = [pl.BlockSpec((B * c, hw), lambda i: (i, 0))]
    in_specs += [pl.BlockSpec(op.shape, lambda i: (0, 0))
                 for op in operands[1:]]

    out_shapes = (
        jax.ShapeDtypeStruct((n * h, w), jnp.float32),
        jax.ShapeDtypeStruct((n * 4, hw), jnp.float32),
        jax.ShapeDtypeStruct((n * h, w), jnp.float32),
        jax.ShapeDtypeStruct((n * 10, hw), jnp.float32),
    )
    out_specs = (
        pl.BlockSpec((B * h, w), lambda i: (i, 0)),
        pl.BlockSpec((B * 4, hw), lambda i: (i, 0)),
        pl.BlockSpec((B * h, w), lambda i: (i, 0)),
        pl.BlockSpec((B * 10, hw), lambda i: (i, 0)),
    )

    cls2, bbox2, obj2, kps2 = pl.pallas_call(
        functools.partial(_yunet_body, H=h, W=w, BC=B * c),
        out_shape=out_shapes,
        grid=(n // B,),
        in_specs=in_specs,
        out_specs=out_specs,
        scratch_shapes=[pltpu.VMEM((9 * B * c, hw), jnp.bfloat16)],
        compiler_params=pltpu.CompilerParams(
            dimension_semantics=("parallel",)),
    )(*operands)

    cls_p = cls2.reshape(n, hw, 1)
    bbox_p = jnp.transpose(bbox2.reshape(n, 4, hw), (0, 2, 1))
    obj_p = obj2.reshape(n, hw, 1)
    kps_p = jnp.transpose(kps2.reshape(n, 10, hw), (0, 2, 1))
    return cls_p, bbox_p, obj_p, kps_p
